# baseline (speedup 1.0000x reference)
import functools

import jax
import jax.numpy as jnp
from jax.experimental import pallas as pl
from jax.experimental.pallas import tpu as pltpu

N, T = 64, 16
NT = N * T
GPT_DIM = 1024
G, C, D = 2, 1024, 256
WA = 5 * 7
HID = 512


def _bin_head_body(x_ref, wbin_ref, bbin_ref, logits_ref, centers_ref):
    x = x_ref[...]
    logits = jnp.dot(x, wbin_ref[...], preferred_element_type=jnp.float32)
    logits = logits + bbin_ref[...]
    logits_ref[...] = logits
    bt = logits.shape[0]
    iota = jax.lax.broadcasted_iota(jnp.int32, (bt, C), 1)
    for g in range(G):
        lg = logits[:, g * C:(g + 1) * C]
        m = jnp.max(lg, axis=1, keepdims=True)
        idx = jnp.where(lg >= m, iota, jnp.int32(2 ** 30))
        centers_ref[:, g] = jnp.min(idx, axis=1)


def _bin_head(xf, W_bin, b_bin):
    BT = 256
    grid = (NT // BT,)
    return pl.pallas_call(
        _bin_head_body,
        grid=grid,
        in_specs=[
            pl.BlockSpec((BT, GPT_DIM), lambda i: (i, 0)),
            pl.BlockSpec((GPT_DIM, G * C), lambda i: (0, 0)),
            pl.BlockSpec((1, G * C), lambda i: (0, 0)),
        ],
        out_specs=[
            pl.BlockSpec((BT, G * C), lambda i: (i, 0)),
            pl.BlockSpec((BT, G), lambda i: (i, 0)),
        ],
        out_shape=[
            jax.ShapeDtypeStruct((NT, G * C), jnp.float32),
            jax.ShapeDtypeStruct((NT, G), jnp.int32),
        ],
    )(xf, W_bin, b_bin.reshape(1, G * C))


def _offsets_body(centers_ref, x_ref, woff_ref, boff_ref, out_ref):
    g = pl.program_id(1)
    x = x_ref[...].reshape(1, GPT_DIM)
    slab = woff_ref[...].reshape(GPT_DIM, WA)
    part = jnp.dot(x, slab, preferred_element_type=jnp.float32)
    part = part + boff_ref[...].reshape(1, WA)

    @pl.when(g == 0)
    def _():
        out_ref[...] = jnp.zeros_like(out_ref)

    out_ref[...] += part.reshape(1, 1, WA)


def _offsets_head(xf, W_off, b_off, centers):
    grid = (NT, G)

    def woff_map(nt, g, c_ref):
        return (0, c_ref[nt, g] + g * C, 0, 0)

    def boff_map(nt, g, c_ref):
        return (c_ref[nt, g] + g * C, 0, 0)

    grid_spec = pltpu.PrefetchScalarGridSpec(
        num_scalar_prefetch=1,
        grid=grid,
        in_specs=[
            pl.BlockSpec((1, 1, GPT_DIM), lambda nt, g, c_ref: (nt, 0, 0)),
            pl.BlockSpec((GPT_DIM, 1, 1, WA), woff_map),
            pl.BlockSpec((1, 1, WA), boff_map),
        ],
        out_specs=pl.BlockSpec((1, 1, WA), lambda nt, g, c_ref: (nt, 0, 0)),
    )
    return pl.pallas_call(
        _offsets_body,
        grid_spec=grid_spec,
        out_shape=jax.ShapeDtypeStruct((NT, 1, WA), jnp.float32),
    )(
        centers,
        xf.reshape(NT, 1, GPT_DIM),
        W_off.reshape(GPT_DIM, G * C, 1, WA),
        b_off.reshape(G * C, 1, WA),
    )


def _decode_body(centers_ref, cb_ref, w1_ref, b1_ref, w2_ref, b2_ref,
                 w3_ref, b3_ref, off_ref, dec_ref, pred_ref):
    centers = centers_ref[...]
    iota = jax.lax.broadcasted_iota(jnp.int32, (NT, G * C), 1)
    gc0 = centers[:, 0:1]
    gc1 = centers[:, 1:2] + C
    onehot = ((iota == gc0) | (iota == gc1)).astype(jnp.float32)
    z = jnp.dot(onehot, cb_ref[...], preferred_element_type=jnp.float32)
    h = jnp.maximum(jnp.dot(z, w1_ref[...], preferred_element_type=jnp.float32)
                    + b1_ref[...], 0.0)
    h = jnp.maximum(jnp.dot(h, w2_ref[...], preferred_element_type=jnp.float32)
                    + b2_ref[...], 0.0)
    dec = jnp.dot(h, w3_ref[...], preferred_element_type=jnp.float32) + b3_ref[...]
    dec_ref[...] = dec
    pred_ref[...] = dec + off_ref[...]


def _decode(centers, codebooks, dec_W1, dec_b1, dec_W2, dec_b2, dec_W3,
            dec_b3, offsets):
    return pl.pallas_call(
        _decode_body,
        out_shape=[
            jax.ShapeDtypeStruct((NT, WA), jnp.float32),
            jax.ShapeDtypeStruct((NT, WA), jnp.float32),
        ],
    )(
        centers,
        codebooks.reshape(G * C, D),
        dec_W1, dec_b1.reshape(1, HID),
        dec_W2, dec_b2.reshape(1, HID),
        dec_W3, dec_b3.reshape(1, WA),
        offsets.reshape(NT, WA),
    )


def kernel(x, W_bin, b_bin, W_off, b_off, codebooks, dec_W1, dec_b1,
           dec_W2, dec_b2, dec_W3, dec_b3):
    xf = x.reshape(NT, GPT_DIM)
    logits, centers = _bin_head(xf, W_bin, b_bin)
    offsets = _offsets_head(xf, W_off, b_off, centers)
    decoded, predicted = _decode(centers, codebooks, dec_W1, dec_b1, dec_W2,
                                 dec_b2, dec_W3, dec_b3, offsets)
    cbet_logits = logits.reshape(NT, G, C)
    predicted_action = predicted.reshape(N, T, WA)
    decoded_action = decoded.reshape(NT, 5, 7)
    return cbet_logits, predicted_action, centers, decoded_action


# speedup vs baseline: 17.1153x; 17.1153x over previous
import functools

import jax
import jax.numpy as jnp
from jax.experimental import pallas as pl
from jax.experimental.pallas import tpu as pltpu

N, T = 64, 16
NT = N * T
GPT_DIM = 1024
G, C, D = 2, 1024, 256
WA = 5 * 7
HID = 512


def _bin_head_body(x_ref, wbin_ref, bbin_ref, logits_ref, centers_ref):
    x = x_ref[...]
    logits = jnp.dot(x, wbin_ref[...], preferred_element_type=jnp.float32)
    logits = logits + bbin_ref[...]
    logits_ref[...] = logits
    bt = logits.shape[0]
    iota = jax.lax.broadcasted_iota(jnp.int32, (bt, C), 1)
    for g in range(G):
        lg = logits[:, g * C:(g + 1) * C]
        m = jnp.max(lg, axis=1, keepdims=True)
        idx = jnp.where(lg >= m, iota, jnp.int32(2 ** 30))
        centers_ref[:, g] = jnp.min(idx, axis=1)


def _bin_head(xf, W_bin, b_bin):
    BT = 256
    grid = (NT // BT,)
    return pl.pallas_call(
        _bin_head_body,
        grid=grid,
        in_specs=[
            pl.BlockSpec((BT, GPT_DIM), lambda i: (i, 0)),
            pl.BlockSpec((GPT_DIM, G * C), lambda i: (0, 0)),
            pl.BlockSpec((1, G * C), lambda i: (0, 0)),
        ],
        out_specs=[
            pl.BlockSpec((BT, G * C), lambda i: (i, 0)),
            pl.BlockSpec((BT, G), lambda i: (i, 0)),
        ],
        out_shape=[
            jax.ShapeDtypeStruct((NT, G * C), jnp.float32),
            jax.ShapeDtypeStruct((NT, G), jnp.int32),
        ],
    )(xf, W_bin, b_bin.reshape(1, G * C))


CBLK = 1280
NBLK = (G * C * WA) // CBLK


def _sweep_body(xbf_ref, w_ref, colcode_ref, centers_ref, out_ref):
    b = pl.program_id(0)
    wblk = w_ref[...].astype(jnp.bfloat16)
    y = jnp.dot(xbf_ref[...], wblk, preferred_element_type=jnp.float32)
    col_code = colcode_ref[...]
    gc0 = centers_ref[:, 0:1]
    gc1 = centers_ref[:, 1:2] + C
    m = ((col_code == gc0) | (col_code == gc1)).astype(jnp.float32)
    jf = jax.lax.broadcasted_iota(jnp.int32, (CBLK, WA), 0).astype(jnp.float32)
    jf = jf + (b * CBLK).astype(jnp.float32)
    jmod = jf - jnp.floor(jf * (1.0 / WA) + 0.5 / WA) * WA
    of = jax.lax.broadcasted_iota(jnp.int32, (CBLK, WA), 1).astype(jnp.float32)
    r = (jmod == of).astype(jnp.float32)
    part = jnp.dot(y * m, r, preferred_element_type=jnp.float32)

    @pl.when(b == 0)
    def _():
        out_ref[...] = jnp.zeros_like(out_ref)

    out_ref[...] += part


def _offsets_head(xbf, W_off, col_code, centers):
    return pl.pallas_call(
        _sweep_body,
        grid=(NBLK,),
        in_specs=[
            pl.BlockSpec((NT, GPT_DIM), lambda b: (0, 0)),
            pl.BlockSpec((GPT_DIM, CBLK), lambda b: (0, b)),
            pl.BlockSpec((1, CBLK), lambda b: (0, b)),
            pl.BlockSpec((NT, G), lambda b: (0, 0)),
        ],
        out_specs=pl.BlockSpec((NT, WA), lambda b: (0, 0)),
        out_shape=jax.ShapeDtypeStruct((NT, WA), jnp.float32),
    )(xbf, W_off, col_code, centers)


def _decode_body(centers_ref, cb_ref, boff_ref, w1_ref, b1_ref, w2_ref,
                 b2_ref, w3_ref, b3_ref, off_ref, dec_ref, pred_ref):
    centers = centers_ref[...]
    iota = jax.lax.broadcasted_iota(jnp.int32, (NT, G * C), 1)
    gc0 = centers[:, 0:1]
    gc1 = centers[:, 1:2] + C
    onehot = ((iota == gc0) | (iota == gc1)).astype(jnp.float32)
    z = jnp.dot(onehot, cb_ref[...], preferred_element_type=jnp.float32)
    h = jnp.maximum(jnp.dot(z, w1_ref[...], preferred_element_type=jnp.float32)
                    + b1_ref[...], 0.0)
    h = jnp.maximum(jnp.dot(h, w2_ref[...], preferred_element_type=jnp.float32)
                    + b2_ref[...], 0.0)
    dec = jnp.dot(h, w3_ref[...], preferred_element_type=jnp.float32) + b3_ref[...]
    boff_sum = jnp.dot(onehot, boff_ref[...], preferred_element_type=jnp.float32)
    dec_ref[...] = dec
    pred_ref[...] = dec + off_ref[...] + boff_sum


def _decode(centers, codebooks, b_off, dec_W1, dec_b1, dec_W2, dec_b2,
            dec_W3, dec_b3, offsets):
    return pl.pallas_call(
        _decode_body,
        out_shape=[
            jax.ShapeDtypeStruct((NT, WA), jnp.float32),
            jax.ShapeDtypeStruct((NT, WA), jnp.float32),
        ],
    )(
        centers,
        codebooks.reshape(G * C, D),
        b_off.reshape(G * C, WA),
        dec_W1, dec_b1.reshape(1, HID),
        dec_W2, dec_b2.reshape(1, HID),
        dec_W3, dec_b3.reshape(1, WA),
        offsets,
    )


def kernel(x, W_bin, b_bin, W_off, b_off, codebooks, dec_W1, dec_b1,
           dec_W2, dec_b2, dec_W3, dec_b3):
    xf = x.reshape(NT, GPT_DIM)
    logits, centers = _bin_head(xf, W_bin, b_bin)
    xbf = xf.astype(jnp.bfloat16)
    col_code = (jnp.arange(G * C * WA, dtype=jnp.int32) // WA).reshape(1, -1)
    offsets = _offsets_head(xbf, W_off, col_code, centers)
    decoded, predicted = _decode(centers, codebooks, b_off, dec_W1, dec_b1,
                                 dec_W2, dec_b2, dec_W3, dec_b3, offsets)
    cbet_logits = logits.reshape(NT, G, C)
    predicted_action = predicted.reshape(N, T, WA)
    decoded_action = decoded.reshape(NT, 5, 7)
    return cbet_logits, predicted_action, centers, decoded_action
